# baseline (speedup 1.0000x reference)
import jax
import jax.numpy as jnp
from jax.experimental import pallas as pl
from jax.experimental.pallas import tpu as pltpu

EPS = 1e-5


def _upsample_onehot(w_lo, w_up, s_lo, s_up, dtype):
    t = jax.lax.broadcasted_iota(jnp.int32, (1, s_up), 1)
    src = (t // w_up // 2) * w_lo + (t % w_up) // 2
    s_idx = jax.lax.broadcasted_iota(jnp.int32, (s_lo, s_up), 0)
    return (s_idx == src).astype(dtype)


def _upsample_chunk(x3, u):
    nb, c, s_lo = x3.shape
    flat = x3.reshape(nb * c, s_lo)
    up = jnp.dot(flat, u, preferred_element_type=jnp.float32)
    return up.astype(jnp.bfloat16).reshape(nb, c, u.shape[1])


def _conv3x3_flat(x, wmat, ww):
    c_in, s = x.shape
    halo = ((ww + 1 + 127) // 128) * 128
    z = jnp.zeros((c_in, halo), x.dtype)
    padded = jnp.concatenate([z, x, z], axis=1)
    col = jax.lax.broadcasted_iota(jnp.int32, (1, s), 1) % ww

    acc = jnp.zeros((wmat.shape[0], s), jnp.float32)
    k = 0
    for dy in (-1, 0, 1):
        for dx in (-1, 0, 1):
            sft = dy * ww + dx
            tap = padded[:, halo + sft: halo + sft + s]
            if dx == -1:
                tap = jnp.where(col >= 1, tap, jnp.zeros_like(tap))
            elif dx == 1:
                tap = jnp.where(col < ww - 1, tap, jnp.zeros_like(tap))
            acc = acc + jnp.dot(wmat[:, k * c_in:(k + 1) * c_in], tap,
                                preferred_element_type=jnp.float32)
            k += 1
    return acc


def _make_stats1_kernel(n_chunks, n_batch, s_lo):
    def body(x_ref, mean_ref, invstd_ref):
        i = pl.program_id(0)

        @pl.when(i == 0)
        def _():
            mean_ref[...] = jnp.zeros_like(mean_ref)
            invstd_ref[...] = jnp.zeros_like(invstd_ref)

        xc = x_ref[...]
        mean_ref[...] += jnp.sum(jnp.sum(xc, axis=2, keepdims=True), axis=0)
        invstd_ref[...] += jnp.sum(jnp.sum(xc * xc, axis=2, keepdims=True),
                                   axis=0)

        @pl.when(i == n_chunks - 1)
        def _():
            cnt = float(n_batch * s_lo)
            mu = mean_ref[...] / cnt
            var = invstd_ref[...] / cnt - mu * mu
            mean_ref[...] = mu
            invstd_ref[...] = jax.lax.rsqrt(var + EPS)

    return body


def _bn1_stats(xf, *, n, c_in, s_lo):
    chunk = 16 if n % 16 == 0 else 1
    n_chunks = n // chunk
    return pl.pallas_call(
        _make_stats1_kernel(n_chunks, n, s_lo),
        grid=(n_chunks,),
        in_specs=[pl.BlockSpec((chunk, c_in, s_lo), lambda i: (i, 0, 0))],
        out_specs=[pl.BlockSpec((c_in, 1), lambda i: (0, 0)),
                   pl.BlockSpec((c_in, 1), lambda i: (0, 0))],
        out_shape=[jax.ShapeDtypeStruct((c_in, 1), jnp.float32),
                   jax.ShapeDtypeStruct((c_in, 1), jnp.float32)],
        compiler_params=pltpu.CompilerParams(
            dimension_semantics=("arbitrary",)),
    )(xf)


def _make_stage1_kernel(h, w, nb):
    w_up = 2 * w

    def body(mean1_ref, invstd1_ref, x_ref, e1_ref, w1_ref, bias1_ref,
             w0_ref, bias0_ref, u_ref, y1_ref, sc_ref, s12_ref):
        c = x_ref.shape[1]
        x3 = x_ref[...]
        g1 = e1_ref[:, :c]
        b1 = e1_ref[:, c:]
        hbn3 = jnp.maximum(
            g1 * ((x3 - mean1_ref[...]) * invstd1_ref[...]) + b1, 0.0)
        hup3 = _upsample_chunk(hbn3.astype(jnp.bfloat16), u_ref[...])
        xb3 = x3.astype(jnp.bfloat16)

        s1_acc = jnp.zeros((s12_ref.shape[1], 1), jnp.float32)
        s2_acc = jnp.zeros((s12_ref.shape[1], 1), jnp.float32)
        for b in range(nb):
            sc = jnp.dot(w0_ref[...], xb3[b],
                         preferred_element_type=jnp.float32) + bias0_ref[...]
            sc_ref[b] = sc.astype(jnp.bfloat16)

            y = _conv3x3_flat(hup3[b], w1_ref[...], w_up) + bias1_ref[...]
            y1_ref[b] = y.astype(jnp.bfloat16)
            s1_acc = s1_acc + jnp.sum(y, axis=1, keepdims=True)
            s2_acc = s2_acc + jnp.sum(y * y, axis=1, keepdims=True)
        s12_ref[0] = jnp.concatenate([s1_acc, s2_acc], axis=1)

    return body


def _stage1(xf, mean1, invstd1, e1, w1mat, bias1, w0mat, bias0, u_lo,
            *, n, c_in, c_out, h, w, nb):
    s_lo = h * w
    s_up = 4 * s_lo
    n_chunks = n // nb
    return pl.pallas_call(
        _make_stage1_kernel(h, w, nb),
        grid=(n_chunks,),
        in_specs=[
            pl.BlockSpec((c_in, 1), lambda i: (0, 0)),
            pl.BlockSpec((c_in, 1), lambda i: (0, 0)),
            pl.BlockSpec((nb, c_in, s_lo), lambda i: (i, 0, 0)),
            pl.BlockSpec((nb, 2 * c_in, 1), lambda i: (i, 0, 0)),
            pl.BlockSpec((c_out, 9 * c_in), lambda i: (0, 0)),
            pl.BlockSpec((c_out, 1), lambda i: (0, 0)),
            pl.BlockSpec((c_out, c_in), lambda i: (0, 0)),
            pl.BlockSpec((c_out, 1), lambda i: (0, 0)),
            pl.BlockSpec((s_lo, s_up), lambda i: (0, 0)),
        ],
        out_specs=[
            pl.BlockSpec((nb, c_out, s_up), lambda i: (i, 0, 0)),
            pl.BlockSpec((nb, c_out, s_lo), lambda i: (i, 0, 0)),
            pl.BlockSpec((1, c_out, 2), lambda i: (i, 0, 0)),
        ],
        out_shape=[
            jax.ShapeDtypeStruct((n, c_out, s_up), jnp.bfloat16),
            jax.ShapeDtypeStruct((n, c_out, s_lo), jnp.bfloat16),
            jax.ShapeDtypeStruct((n_chunks, c_out, 2), jnp.float32),
        ],
        compiler_params=pltpu.CompilerParams(
            dimension_semantics=("parallel",)),
    )(mean1, invstd1, xf, e1, w1mat, bias1, w0mat, bias0, u_lo)


def _make_stage2_kernel(n_batch, h, w, nb):
    w_up = 2 * w
    s_up = 4 * h * w
    cnt2 = float(n_batch * s_up)

    def body(s12_ref, y1_ref, e2_ref, w2_ref, bias2_ref,
             sc_ref, u_ref, out_ref):
        c = y1_ref.shape[1]
        s12 = jnp.sum(s12_ref[...], axis=0)
        mu = s12[:, :1] / cnt2
        ex2 = s12[:, 1:] / cnt2
        iv = jax.lax.rsqrt(ex2 - mu * mu + EPS)

        y3 = y1_ref[...].astype(jnp.float32)
        g2 = e2_ref[:, :c]
        b2 = e2_ref[:, c:]
        z3 = jnp.maximum(g2 * ((y3 - mu) * iv) + b2, 0.0).astype(jnp.bfloat16)
        res3 = _upsample_chunk(sc_ref[...], u_ref[...]).astype(jnp.float32)
        for b in range(nb):
            y = _conv3x3_flat(z3[b], w2_ref[...], w_up) + bias2_ref[...]
            out_ref[b] = y + res3[b]

    return body


def _stage2(y1, sc, s12, e2, w2mat, bias2, u_lo, *, n, c_out, h, w, nb):
    s_lo = h * w
    s_up = 4 * s_lo
    n_chunks = n // nb
    n_stat_chunks = s12.shape[0]
    return pl.pallas_call(
        _make_stage2_kernel(n, h, w, nb),
        grid=(n_chunks,),
        in_specs=[
            pl.BlockSpec((n_stat_chunks, c_out, 2), lambda i: (0, 0, 0)),
            pl.BlockSpec((nb, c_out, s_up), lambda i: (i, 0, 0)),
            pl.BlockSpec((nb, 2 * c_out, 1), lambda i: (i, 0, 0)),
            pl.BlockSpec((c_out, 9 * c_out), lambda i: (0, 0)),
            pl.BlockSpec((c_out, 1), lambda i: (0, 0)),
            pl.BlockSpec((nb, c_out, s_lo), lambda i: (i, 0, 0)),
            pl.BlockSpec((s_lo, s_up), lambda i: (0, 0)),
        ],
        out_specs=pl.BlockSpec((nb, c_out, s_up), lambda i: (i, 0, 0)),
        out_shape=jax.ShapeDtypeStruct((n, c_out, s_up), jnp.float32),
        compiler_params=pltpu.CompilerParams(
            dimension_semantics=("parallel",)),
    )(s12, y1, e2, w2mat, bias2, sc, u_lo)


def kernel(x, labels, embed1, embed2, w1, b1, w2, b2, w0, b0):
    n, c_in, h, w = x.shape
    c_out = w1.shape[0]
    s_lo = h * w
    s_up = 4 * s_lo

    xf = x.reshape(n, c_in, s_lo)

    e1 = embed1[labels].reshape(n, 2 * c_in, 1)
    e2 = embed2[labels].reshape(n, 2 * c_out, 1)

    w1mat = (jnp.transpose(w1, (0, 2, 3, 1)).reshape(c_out, 9 * c_in)
             .astype(jnp.bfloat16))
    w2mat = (jnp.transpose(w2, (0, 2, 3, 1)).reshape(c_out, 9 * c_out)
             .astype(jnp.bfloat16))
    w0mat = w0.reshape(c_out, c_in).astype(jnp.bfloat16)
    bias1 = b1.reshape(c_out, 1)
    bias2 = b2.reshape(c_out, 1)
    bias0 = b0.reshape(c_out, 1)

    u_lo = _upsample_onehot(w, 2 * w, s_lo, s_up, jnp.bfloat16)

    nb = 4 if n % 4 == 0 else 1
    mean1, invstd1 = _bn1_stats(xf, n=n, c_in=c_in, s_lo=s_lo)
    y1, sc, s12 = _stage1(xf, mean1, invstd1, e1, w1mat, bias1,
                          w0mat, bias0, u_lo,
                          n=n, c_in=c_in, c_out=c_out, h=h, w=w, nb=nb)
    out = _stage2(y1, sc, s12, e2, w2mat, bias2, u_lo,
                  n=n, c_out=c_out, h=h, w=w, nb=nb)
    return out.reshape(n, c_out, 2 * h, 2 * w)


# speedup vs baseline: 1.1198x; 1.1198x over previous
import jax
import jax.numpy as jnp
from jax.experimental import pallas as pl
from jax.experimental.pallas import tpu as pltpu

EPS = 1e-5


def _upsample_onehot(w_lo, w_up, s_lo, s_up, dtype):
    t = jax.lax.broadcasted_iota(jnp.int32, (1, s_up), 1)
    src = (t // w_up // 2) * w_lo + (t % w_up) // 2
    s_idx = jax.lax.broadcasted_iota(jnp.int32, (s_lo, s_up), 0)
    return (s_idx == src).astype(dtype)


def _upsample_chunk(x3, u):
    nb, c, s_lo = x3.shape
    flat = x3.reshape(nb * c, s_lo)
    up = jnp.dot(flat, u, preferred_element_type=jnp.float32)
    return up.astype(jnp.bfloat16).reshape(nb, c, u.shape[1])


def _conv3x3_flat(x, wmat, ww):
    c_in, s = x.shape
    halo = ((ww + 1 + 127) // 128) * 128
    z = jnp.zeros((c_in, halo), x.dtype)
    padded = jnp.concatenate([z, x, z], axis=1)
    col = jax.lax.broadcasted_iota(jnp.int32, (1, s), 1) % ww

    acc = jnp.zeros((wmat.shape[0], s), jnp.float32)
    k = 0
    for dy in (-1, 0, 1):
        for dx in (-1, 0, 1):
            sft = dy * ww + dx
            tap = padded[:, halo + sft: halo + sft + s]
            if dx == -1:
                tap = jnp.where(col >= 1, tap, jnp.zeros_like(tap))
            elif dx == 1:
                tap = jnp.where(col < ww - 1, tap, jnp.zeros_like(tap))
            acc = acc + jnp.dot(wmat[:, k * c_in:(k + 1) * c_in], tap,
                                preferred_element_type=jnp.float32)
            k += 1
    return acc


def _make_stats1_kernel(n_chunks, n_batch, s_lo):
    def body(x_ref, mean_ref, invstd_ref):
        i = pl.program_id(0)

        @pl.when(i == 0)
        def _():
            mean_ref[...] = jnp.zeros_like(mean_ref)
            invstd_ref[...] = jnp.zeros_like(invstd_ref)

        xc = x_ref[...]
        mean_ref[...] += jnp.sum(jnp.sum(xc, axis=2, keepdims=True), axis=0)
        invstd_ref[...] += jnp.sum(jnp.sum(xc * xc, axis=2, keepdims=True),
                                   axis=0)

        @pl.when(i == n_chunks - 1)
        def _():
            cnt = float(n_batch * s_lo)
            mu = mean_ref[...] / cnt
            var = invstd_ref[...] / cnt - mu * mu
            mean_ref[...] = mu
            invstd_ref[...] = jax.lax.rsqrt(var + EPS)

    return body


def _bn1_stats(xf, *, n, c_in, s_lo):
    chunk = 16 if n % 16 == 0 else 1
    n_chunks = n // chunk
    return pl.pallas_call(
        _make_stats1_kernel(n_chunks, n, s_lo),
        grid=(n_chunks,),
        in_specs=[pl.BlockSpec((chunk, c_in, s_lo), lambda i: (i, 0, 0))],
        out_specs=[pl.BlockSpec((c_in, 1), lambda i: (0, 0)),
                   pl.BlockSpec((c_in, 1), lambda i: (0, 0))],
        out_shape=[jax.ShapeDtypeStruct((c_in, 1), jnp.float32),
                   jax.ShapeDtypeStruct((c_in, 1), jnp.float32)],
        compiler_params=pltpu.CompilerParams(
            dimension_semantics=("arbitrary",)),
    )(xf)


def _make_stage1_kernel(h, w, nb):
    w_up = 2 * w

    def body(mean1_ref, invstd1_ref, x_ref, e1_ref, w1_ref, bias1_ref,
             w0_ref, bias0_ref, u_ref, y1_ref, sc_ref, s12_ref):
        c = x_ref.shape[1]
        x3 = x_ref[...]
        a1 = e1_ref[:, :c] * invstd1_ref[...]
        b1 = e1_ref[:, c:] - a1 * mean1_ref[...]
        hbn3 = jnp.maximum(a1 * x3 + b1, 0.0)
        hup3 = _upsample_chunk(hbn3.astype(jnp.bfloat16), u_ref[...])
        xb3 = x3.astype(jnp.bfloat16)

        s1_acc = jnp.zeros((s12_ref.shape[1], 1), jnp.float32)
        s2_acc = jnp.zeros((s12_ref.shape[1], 1), jnp.float32)
        for b in range(nb):
            sc = jnp.dot(w0_ref[...], xb3[b],
                         preferred_element_type=jnp.float32) + bias0_ref[...]
            sc_ref[b] = sc.astype(jnp.bfloat16)

            y = _conv3x3_flat(hup3[b], w1_ref[...], w_up) + bias1_ref[...]
            y1_ref[b] = y.astype(jnp.bfloat16)
            s1_acc = s1_acc + jnp.sum(y, axis=1, keepdims=True)
            s2_acc = s2_acc + jnp.sum(y * y, axis=1, keepdims=True)
        s12_ref[0] = jnp.concatenate([s1_acc, s2_acc], axis=1)

    return body


def _stage1(xf, mean1, invstd1, e1, w1mat, bias1, w0mat, bias0, u_lo,
            *, n, c_in, c_out, h, w, nb):
    s_lo = h * w
    s_up = 4 * s_lo
    n_chunks = n // nb
    return pl.pallas_call(
        _make_stage1_kernel(h, w, nb),
        grid=(n_chunks,),
        in_specs=[
            pl.BlockSpec((c_in, 1), lambda i: (0, 0)),
            pl.BlockSpec((c_in, 1), lambda i: (0, 0)),
            pl.BlockSpec((nb, c_in, s_lo), lambda i: (i, 0, 0)),
            pl.BlockSpec((nb, 2 * c_in, 1), lambda i: (i, 0, 0)),
            pl.BlockSpec((c_out, 9 * c_in), lambda i: (0, 0)),
            pl.BlockSpec((c_out, 1), lambda i: (0, 0)),
            pl.BlockSpec((c_out, c_in), lambda i: (0, 0)),
            pl.BlockSpec((c_out, 1), lambda i: (0, 0)),
            pl.BlockSpec((s_lo, s_up), lambda i: (0, 0)),
        ],
        out_specs=[
            pl.BlockSpec((nb, c_out, s_up), lambda i: (i, 0, 0)),
            pl.BlockSpec((nb, c_out, s_lo), lambda i: (i, 0, 0)),
            pl.BlockSpec((1, c_out, 2), lambda i: (i, 0, 0)),
        ],
        out_shape=[
            jax.ShapeDtypeStruct((n, c_out, s_up), jnp.bfloat16),
            jax.ShapeDtypeStruct((n, c_out, s_lo), jnp.bfloat16),
            jax.ShapeDtypeStruct((n_chunks, c_out, 2), jnp.float32),
        ],
        compiler_params=pltpu.CompilerParams(
            dimension_semantics=("parallel",)),
    )(mean1, invstd1, xf, e1, w1mat, bias1, w0mat, bias0, u_lo)


def _make_stage2_kernel(n_batch, h, w, nb):
    w_up = 2 * w
    s_up = 4 * h * w
    cnt2 = float(n_batch * s_up)

    def body(s12_ref, y1_ref, e2_ref, w2_ref, bias2_ref,
             sc_ref, u_ref, out_ref):
        c = y1_ref.shape[1]
        s12 = jnp.sum(s12_ref[...], axis=0)
        mu = s12[:, :1] / cnt2
        ex2 = s12[:, 1:] / cnt2
        iv = jax.lax.rsqrt(ex2 - mu * mu + EPS)

        a2 = e2_ref[:, :c] * iv
        b2 = e2_ref[:, c:] - a2 * mu
        res3 = _upsample_chunk(sc_ref[...], u_ref[...])
        for b in range(nb):
            yb = y1_ref[b].astype(jnp.float32)
            z = jnp.maximum(a2[b] * yb + b2[b], 0.0).astype(jnp.bfloat16)
            y = _conv3x3_flat(z, w2_ref[...], w_up) + bias2_ref[...]
            out_ref[b] = y + res3[b].astype(jnp.float32)

    return body


def _stage2(y1, sc, s12, e2, w2mat, bias2, u_lo, *, n, c_out, h, w, nb):
    s_lo = h * w
    s_up = 4 * s_lo
    n_chunks = n // nb
    n_stat_chunks = s12.shape[0]
    return pl.pallas_call(
        _make_stage2_kernel(n, h, w, nb),
        grid=(n_chunks,),
        in_specs=[
            pl.BlockSpec((n_stat_chunks, c_out, 2), lambda i: (0, 0, 0)),
            pl.BlockSpec((nb, c_out, s_up), lambda i: (i, 0, 0)),
            pl.BlockSpec((nb, 2 * c_out, 1), lambda i: (i, 0, 0)),
            pl.BlockSpec((c_out, 9 * c_out), lambda i: (0, 0)),
            pl.BlockSpec((c_out, 1), lambda i: (0, 0)),
            pl.BlockSpec((nb, c_out, s_lo), lambda i: (i, 0, 0)),
            pl.BlockSpec((s_lo, s_up), lambda i: (0, 0)),
        ],
        out_specs=pl.BlockSpec((nb, c_out, s_up), lambda i: (i, 0, 0)),
        out_shape=jax.ShapeDtypeStruct((n, c_out, s_up), jnp.float32),
        compiler_params=pltpu.CompilerParams(
            dimension_semantics=("parallel",)),
    )(s12, y1, e2, w2mat, bias2, sc, u_lo)


def kernel(x, labels, embed1, embed2, w1, b1, w2, b2, w0, b0):
    n, c_in, h, w = x.shape
    c_out = w1.shape[0]
    s_lo = h * w
    s_up = 4 * s_lo

    xf = x.reshape(n, c_in, s_lo)

    e1 = embed1[labels].reshape(n, 2 * c_in, 1)
    e2 = embed2[labels].reshape(n, 2 * c_out, 1)

    w1mat = (jnp.transpose(w1, (0, 2, 3, 1)).reshape(c_out, 9 * c_in)
             .astype(jnp.bfloat16))
    w2mat = (jnp.transpose(w2, (0, 2, 3, 1)).reshape(c_out, 9 * c_out)
             .astype(jnp.bfloat16))
    w0mat = w0.reshape(c_out, c_in).astype(jnp.bfloat16)
    bias1 = b1.reshape(c_out, 1)
    bias2 = b2.reshape(c_out, 1)
    bias0 = b0.reshape(c_out, 1)

    u_lo = _upsample_onehot(w, 2 * w, s_lo, s_up, jnp.bfloat16)

    nb = 8 if n % 8 == 0 else 1
    mean1, invstd1 = _bn1_stats(xf, n=n, c_in=c_in, s_lo=s_lo)
    y1, sc, s12 = _stage1(xf, mean1, invstd1, e1, w1mat, bias1,
                          w0mat, bias0, u_lo,
                          n=n, c_in=c_in, c_out=c_out, h=h, w=w, nb=nb)
    out = _stage2(y1, sc, s12, e2, w2mat, bias2, u_lo,
                  n=n, c_out=c_out, h=h, w=w, nb=nb)
    return out.reshape(n, c_out, 2 * h, 2 * w)


# speedup vs baseline: 1.1227x; 1.0026x over previous
import jax
import jax.numpy as jnp
from jax.experimental import pallas as pl
from jax.experimental.pallas import tpu as pltpu

EPS = 1e-5


def _upsample_onehot(w_lo, w_up, s_lo, s_up, dtype):
    t = jax.lax.broadcasted_iota(jnp.int32, (1, s_up), 1)
    src = (t // w_up // 2) * w_lo + (t % w_up) // 2
    s_idx = jax.lax.broadcasted_iota(jnp.int32, (s_lo, s_up), 0)
    return (s_idx == src).astype(dtype)


def _upsample_chunk(x3, u):
    nb, c, s_lo = x3.shape
    flat = x3.reshape(nb * c, s_lo)
    up = jnp.dot(flat, u, preferred_element_type=jnp.float32)
    return up.astype(jnp.bfloat16).reshape(nb, c, u.shape[1])


def _conv3x3_flat(x, wmat, ww):
    c_in, s = x.shape
    halo = ((ww + 1 + 127) // 128) * 128
    z = jnp.zeros((c_in, halo), x.dtype)
    padded = jnp.concatenate([z, x, z], axis=1)
    col = jax.lax.broadcasted_iota(jnp.int32, (1, s + 2 * halo), 1) - halo
    colw = col % ww
    zp = jnp.zeros_like(padded)
    pad_l = jnp.where(colw < ww - 1, padded, zp)
    pad_r = jnp.where(colw >= 1, padded, zp)
    src = {-1: pad_l, 0: padded, 1: pad_r}

    acc = jnp.zeros((wmat.shape[0], s), jnp.float32)
    k = 0
    for dy in (-1, 0, 1):
        for dx in (-1, 0, 1):
            sft = dy * ww + dx
            tap = src[dx][:, halo + sft: halo + sft + s]
            acc = acc + jnp.dot(wmat[:, k * c_in:(k + 1) * c_in], tap,
                                preferred_element_type=jnp.float32)
            k += 1
    return acc


def _make_stats1_kernel(n_chunks, n_batch, s_lo):
    def body(x_ref, mean_ref, invstd_ref):
        i = pl.program_id(0)

        @pl.when(i == 0)
        def _():
            mean_ref[...] = jnp.zeros_like(mean_ref)
            invstd_ref[...] = jnp.zeros_like(invstd_ref)

        xc = x_ref[...]
        mean_ref[...] += jnp.sum(jnp.sum(xc, axis=2, keepdims=True), axis=0)
        invstd_ref[...] += jnp.sum(jnp.sum(xc * xc, axis=2, keepdims=True),
                                   axis=0)

        @pl.when(i == n_chunks - 1)
        def _():
            cnt = float(n_batch * s_lo)
            mu = mean_ref[...] / cnt
            var = invstd_ref[...] / cnt - mu * mu
            mean_ref[...] = mu
            invstd_ref[...] = jax.lax.rsqrt(var + EPS)

    return body


def _bn1_stats(xf, *, n, c_in, s_lo):
    chunk = 16 if n % 16 == 0 else 1
    n_chunks = n // chunk
    return pl.pallas_call(
        _make_stats1_kernel(n_chunks, n, s_lo),
        grid=(n_chunks,),
        in_specs=[pl.BlockSpec((chunk, c_in, s_lo), lambda i: (i, 0, 0))],
        out_specs=[pl.BlockSpec((c_in, 1), lambda i: (0, 0)),
                   pl.BlockSpec((c_in, 1), lambda i: (0, 0))],
        out_shape=[jax.ShapeDtypeStruct((c_in, 1), jnp.float32),
                   jax.ShapeDtypeStruct((c_in, 1), jnp.float32)],
        compiler_params=pltpu.CompilerParams(
            dimension_semantics=("arbitrary",)),
    )(xf)


def _make_stage1_kernel(h, w, nb):
    w_up = 2 * w

    def body(mean1_ref, invstd1_ref, x_ref, e1_ref, w1_ref, bias1_ref,
             w0_ref, bias0_ref, u_ref, y1_ref, sc_ref, s12_ref):
        c = x_ref.shape[1]
        x3 = x_ref[...]
        a1 = e1_ref[:, :c] * invstd1_ref[...]
        b1 = e1_ref[:, c:] - a1 * mean1_ref[...]
        hbn3 = jnp.maximum(a1 * x3 + b1, 0.0)
        hup3 = _upsample_chunk(hbn3.astype(jnp.bfloat16), u_ref[...])
        xb3 = x3.astype(jnp.bfloat16)

        s1_acc = jnp.zeros((s12_ref.shape[1], 1), jnp.float32)
        s2_acc = jnp.zeros((s12_ref.shape[1], 1), jnp.float32)
        for b in range(nb):
            sc = jnp.dot(w0_ref[...], xb3[b],
                         preferred_element_type=jnp.float32) + bias0_ref[...]
            sc_ref[b] = sc.astype(jnp.bfloat16)

            y = _conv3x3_flat(hup3[b], w1_ref[...], w_up) + bias1_ref[...]
            y1_ref[b] = y.astype(jnp.bfloat16)
            s1_acc = s1_acc + jnp.sum(y, axis=1, keepdims=True)
            s2_acc = s2_acc + jnp.sum(y * y, axis=1, keepdims=True)
        s12_ref[0] = jnp.concatenate([s1_acc, s2_acc], axis=1)

    return body


def _stage1(xf, mean1, invstd1, e1, w1mat, bias1, w0mat, bias0, u_lo,
            *, n, c_in, c_out, h, w, nb):
    s_lo = h * w
    s_up = 4 * s_lo
    n_chunks = n // nb
    return pl.pallas_call(
        _make_stage1_kernel(h, w, nb),
        grid=(n_chunks,),
        in_specs=[
            pl.BlockSpec((c_in, 1), lambda i: (0, 0)),
            pl.BlockSpec((c_in, 1), lambda i: (0, 0)),
            pl.BlockSpec((nb, c_in, s_lo), lambda i: (i, 0, 0)),
            pl.BlockSpec((nb, 2 * c_in, 1), lambda i: (i, 0, 0)),
            pl.BlockSpec((c_out, 9 * c_in), lambda i: (0, 0)),
            pl.BlockSpec((c_out, 1), lambda i: (0, 0)),
            pl.BlockSpec((c_out, c_in), lambda i: (0, 0)),
            pl.BlockSpec((c_out, 1), lambda i: (0, 0)),
            pl.BlockSpec((s_lo, s_up), lambda i: (0, 0)),
        ],
        out_specs=[
            pl.BlockSpec((nb, c_out, s_up), lambda i: (i, 0, 0)),
            pl.BlockSpec((nb, c_out, s_lo), lambda i: (i, 0, 0)),
            pl.BlockSpec((1, c_out, 2), lambda i: (i, 0, 0)),
        ],
        out_shape=[
            jax.ShapeDtypeStruct((n, c_out, s_up), jnp.bfloat16),
            jax.ShapeDtypeStruct((n, c_out, s_lo), jnp.bfloat16),
            jax.ShapeDtypeStruct((n_chunks, c_out, 2), jnp.float32),
        ],
        compiler_params=pltpu.CompilerParams(
            dimension_semantics=("parallel",)),
    )(mean1, invstd1, xf, e1, w1mat, bias1, w0mat, bias0, u_lo)


def _make_stage2_kernel(n_batch, h, w, nb):
    w_up = 2 * w
    s_up = 4 * h * w
    cnt2 = float(n_batch * s_up)

    def body(s12_ref, y1_ref, e2_ref, w2_ref, bias2_ref,
             sc_ref, u_ref, out_ref):
        c = y1_ref.shape[1]
        s12 = jnp.sum(s12_ref[...], axis=0)
        mu = s12[:, :1] / cnt2
        ex2 = s12[:, 1:] / cnt2
        iv = jax.lax.rsqrt(ex2 - mu * mu + EPS)

        a2 = e2_ref[:, :c] * iv
        b2 = e2_ref[:, c:] - a2 * mu
        res3 = _upsample_chunk(sc_ref[...], u_ref[...])
        for b in range(nb):
            yb = y1_ref[b].astype(jnp.float32)
            z = jnp.maximum(a2[b] * yb + b2[b], 0.0).astype(jnp.bfloat16)
            y = _conv3x3_flat(z, w2_ref[...], w_up) + bias2_ref[...]
            out_ref[b] = y + res3[b].astype(jnp.float32)

    return body


def _stage2(y1, sc, s12, e2, w2mat, bias2, u_lo, *, n, c_out, h, w, nb):
    s_lo = h * w
    s_up = 4 * s_lo
    n_chunks = n // nb
    n_stat_chunks = s12.shape[0]
    return pl.pallas_call(
        _make_stage2_kernel(n, h, w, nb),
        grid=(n_chunks,),
        in_specs=[
            pl.BlockSpec((n_stat_chunks, c_out, 2), lambda i: (0, 0, 0)),
            pl.BlockSpec((nb, c_out, s_up), lambda i: (i, 0, 0)),
            pl.BlockSpec((nb, 2 * c_out, 1), lambda i: (i, 0, 0)),
            pl.BlockSpec((c_out, 9 * c_out), lambda i: (0, 0)),
            pl.BlockSpec((c_out, 1), lambda i: (0, 0)),
            pl.BlockSpec((nb, c_out, s_lo), lambda i: (i, 0, 0)),
            pl.BlockSpec((s_lo, s_up), lambda i: (0, 0)),
        ],
        out_specs=pl.BlockSpec((nb, c_out, s_up), lambda i: (i, 0, 0)),
        out_shape=jax.ShapeDtypeStruct((n, c_out, s_up), jnp.float32),
        compiler_params=pltpu.CompilerParams(
            dimension_semantics=("parallel",)),
    )(s12, y1, e2, w2mat, bias2, sc, u_lo)


def kernel(x, labels, embed1, embed2, w1, b1, w2, b2, w0, b0):
    n, c_in, h, w = x.shape
    c_out = w1.shape[0]
    s_lo = h * w
    s_up = 4 * s_lo

    xf = x.reshape(n, c_in, s_lo)

    e1 = embed1[labels].reshape(n, 2 * c_in, 1)
    e2 = embed2[labels].reshape(n, 2 * c_out, 1)

    w1mat = (jnp.transpose(w1, (0, 2, 3, 1)).reshape(c_out, 9 * c_in)
             .astype(jnp.bfloat16))
    w2mat = (jnp.transpose(w2, (0, 2, 3, 1)).reshape(c_out, 9 * c_out)
             .astype(jnp.bfloat16))
    w0mat = w0.reshape(c_out, c_in).astype(jnp.bfloat16)
    bias1 = b1.reshape(c_out, 1)
    bias2 = b2.reshape(c_out, 1)
    bias0 = b0.reshape(c_out, 1)

    u_lo = _upsample_onehot(w, 2 * w, s_lo, s_up, jnp.bfloat16)

    nb = 8 if n % 8 == 0 else 1
    mean1, invstd1 = _bn1_stats(xf, n=n, c_in=c_in, s_lo=s_lo)
    y1, sc, s12 = _stage1(xf, mean1, invstd1, e1, w1mat, bias1,
                          w0mat, bias0, u_lo,
                          n=n, c_in=c_in, c_out=c_out, h=h, w=w, nb=nb)
    out = _stage2(y1, sc, s12, e2, w2mat, bias2, u_lo,
                  n=n, c_out=c_out, h=h, w=w, nb=nb)
    return out.reshape(n, c_out, 2 * h, 2 * w)


# speedup vs baseline: 1.1261x; 1.0030x over previous
import jax
import jax.numpy as jnp
from jax.experimental import pallas as pl
from jax.experimental.pallas import tpu as pltpu

EPS = 1e-5


def _upsample_onehot(w_lo, w_up, s_lo, s_up, dtype):
    t = jax.lax.broadcasted_iota(jnp.int32, (1, s_up), 1)
    src = (t // w_up // 2) * w_lo + (t % w_up) // 2
    s_idx = jax.lax.broadcasted_iota(jnp.int32, (s_lo, s_up), 0)
    return (s_idx == src).astype(dtype)


def _upsample_chunk(x3, u):
    nb, c, s_lo = x3.shape
    flat = x3.reshape(nb * c, s_lo)
    up = jnp.dot(flat, u, preferred_element_type=jnp.float32)
    return up.astype(jnp.bfloat16).reshape(nb, c, u.shape[1])


def _conv3x3_flat(x, wmat, ww):
    c_in, s = x.shape
    halo = ((ww + 1 + 127) // 128) * 128
    z = jnp.zeros((c_in, halo), x.dtype)
    padded = jnp.concatenate([z, x, z], axis=1)
    col = jax.lax.broadcasted_iota(jnp.int32, (1, s + 2 * halo), 1) - halo
    colw = col % ww
    zp = jnp.zeros_like(padded)
    pad_l = jnp.where(colw < ww - 1, padded, zp)
    pad_r = jnp.where(colw >= 1, padded, zp)
    src = {-1: pad_l, 0: padded, 1: pad_r}

    acc = jnp.zeros((wmat.shape[0], s), jnp.float32)
    k = 0
    for dy in (-1, 0, 1):
        for dx in (-1, 0, 1):
            sft = dy * ww + dx
            tap = src[dx][:, halo + sft: halo + sft + s]
            acc = acc + jnp.dot(wmat[:, k * c_in:(k + 1) * c_in], tap,
                                preferred_element_type=jnp.float32)
            k += 1
    return acc


def _make_stats1_kernel(n_chunks, n_batch, s_lo):
    def body(x_ref, mean_ref, invstd_ref):
        i = pl.program_id(0)

        @pl.when(i == 0)
        def _():
            mean_ref[...] = jnp.zeros_like(mean_ref)
            invstd_ref[...] = jnp.zeros_like(invstd_ref)

        xc = x_ref[...]
        mean_ref[...] += jnp.sum(jnp.sum(xc, axis=2, keepdims=True), axis=0)
        invstd_ref[...] += jnp.sum(jnp.sum(xc * xc, axis=2, keepdims=True),
                                   axis=0)

        @pl.when(i == n_chunks - 1)
        def _():
            cnt = float(n_batch * s_lo)
            mu = mean_ref[...] / cnt
            var = invstd_ref[...] / cnt - mu * mu
            mean_ref[...] = mu
            invstd_ref[...] = jax.lax.rsqrt(var + EPS)

    return body


def _bn1_stats(xf, *, n, c_in, s_lo):
    chunk = 16 if n % 16 == 0 else 1
    n_chunks = n // chunk
    return pl.pallas_call(
        _make_stats1_kernel(n_chunks, n, s_lo),
        grid=(n_chunks,),
        in_specs=[pl.BlockSpec((chunk, c_in, s_lo), lambda i: (i, 0, 0))],
        out_specs=[pl.BlockSpec((c_in, 1), lambda i: (0, 0)),
                   pl.BlockSpec((c_in, 1), lambda i: (0, 0))],
        out_shape=[jax.ShapeDtypeStruct((c_in, 1), jnp.float32),
                   jax.ShapeDtypeStruct((c_in, 1), jnp.float32)],
        compiler_params=pltpu.CompilerParams(
            dimension_semantics=("arbitrary",)),
    )(xf)


def _make_fused_kernel(n, h, w, nb1, nb2):
    w_up = 2 * w
    s_up = 4 * h * w
    n1 = n // nb1
    cnt2 = float(n * s_up)

    def body(mean1_ref, invstd1_ref, xa_ref, e1_ref, xb_ref, e2_ref,
             w1_ref, bias1_ref, w0_ref, bias0_ref, w2_ref, bias2_ref, u_ref,
             out_ref, y1_scr, s1_scr, s2_scr, mu2_scr, iv2_scr):
        i = pl.program_id(0)
        c = xa_ref.shape[1]

        @pl.when(i == 0)
        def _():
            s1_scr[...] = jnp.zeros_like(s1_scr)
            s2_scr[...] = jnp.zeros_like(s2_scr)

        @pl.when(i < n1)
        def _phase_a():
            x3 = xa_ref[...]
            a1 = e1_ref[:, :c] * invstd1_ref[...]
            b1 = e1_ref[:, c:] - a1 * mean1_ref[...]
            hbn3 = jnp.maximum(a1 * x3 + b1, 0.0).astype(jnp.bfloat16)
            hup3 = _upsample_chunk(hbn3, u_ref[...])

            s1_acc = jnp.zeros_like(s1_scr[...])
            s2_acc = jnp.zeros_like(s2_scr[...])
            for b in range(nb1):
                y = _conv3x3_flat(hup3[b], w1_ref[...], w_up) + bias1_ref[...]
                y1_scr[i * nb1 + b] = y.astype(jnp.bfloat16)
                s1_acc = s1_acc + jnp.sum(y, axis=1, keepdims=True)
                s2_acc = s2_acc + jnp.sum(y * y, axis=1, keepdims=True)
            s1_scr[...] += s1_acc
            s2_scr[...] += s2_acc

        @pl.when(i == n1)
        def _finalize():
            mu = s1_scr[...] / cnt2
            var = s2_scr[...] / cnt2 - mu * mu
            mu2_scr[...] = mu
            iv2_scr[...] = jax.lax.rsqrt(var + EPS)

        @pl.when(i >= n1)
        def _phase_b():
            j = i - n1
            xb3 = xb_ref[...].astype(jnp.bfloat16)
            sc3 = []
            for b in range(nb2):
                sc = (jnp.dot(w0_ref[...], xb3[b],
                              preferred_element_type=jnp.float32)
                      + bias0_ref[...])
                sc3.append(sc.astype(jnp.bfloat16))
            res3 = _upsample_chunk(jnp.stack(sc3), u_ref[...])

            mu = mu2_scr[...]
            iv = iv2_scr[...]
            a2 = e2_ref[:, :c] * iv
            b2 = e2_ref[:, c:] - a2 * mu
            for b in range(nb2):
                yb = y1_scr[j * nb2 + b].astype(jnp.float32)
                z = jnp.maximum(a2[b] * yb + b2[b], 0.0).astype(jnp.bfloat16)
                y = _conv3x3_flat(z, w2_ref[...], w_up) + bias2_ref[...]
                out_ref[b] = y + res3[b].astype(jnp.float32)

    return body


def _fused(xf, mean1, invstd1, e1, e2, w1mat, bias1, w0mat, bias0,
           w2mat, bias2, u_lo, *, n, c_in, c_out, h, w, nb1, nb2):
    s_lo = h * w
    s_up = 4 * s_lo
    n1 = n // nb1
    n2 = n // nb2

    def _a_idx(i):
        return jnp.minimum(i, n1 - 1)

    def _b_idx(i):
        return jnp.maximum(i - n1, 0)

    return pl.pallas_call(
        _make_fused_kernel(n, h, w, nb1, nb2),
        grid=(n1 + n2,),
        in_specs=[
            pl.BlockSpec((c_in, 1), lambda i: (0, 0)),
            pl.BlockSpec((c_in, 1), lambda i: (0, 0)),
            pl.BlockSpec((nb1, c_in, s_lo), lambda i: (_a_idx(i), 0, 0)),
            pl.BlockSpec((nb1, 2 * c_in, 1), lambda i: (_a_idx(i), 0, 0)),
            pl.BlockSpec((nb2, c_in, s_lo), lambda i: (_b_idx(i), 0, 0)),
            pl.BlockSpec((nb2, 2 * c_out, 1), lambda i: (_b_idx(i), 0, 0)),
            pl.BlockSpec((c_out, 9 * c_in), lambda i: (0, 0)),
            pl.BlockSpec((c_out, 1), lambda i: (0, 0)),
            pl.BlockSpec((c_out, c_in), lambda i: (0, 0)),
            pl.BlockSpec((c_out, 1), lambda i: (0, 0)),
            pl.BlockSpec((c_out, 9 * c_out), lambda i: (0, 0)),
            pl.BlockSpec((c_out, 1), lambda i: (0, 0)),
            pl.BlockSpec((s_lo, s_up), lambda i: (0, 0)),
        ],
        out_specs=pl.BlockSpec((nb2, c_out, s_up), lambda i: (_b_idx(i), 0, 0)),
        out_shape=jax.ShapeDtypeStruct((n, c_out, s_up), jnp.float32),
        scratch_shapes=[
            pltpu.VMEM((n, c_out, s_up), jnp.bfloat16),
            pltpu.VMEM((c_out, 1), jnp.float32),
            pltpu.VMEM((c_out, 1), jnp.float32),
            pltpu.VMEM((c_out, 1), jnp.float32),
            pltpu.VMEM((c_out, 1), jnp.float32),
        ],
        compiler_params=pltpu.CompilerParams(
            dimension_semantics=("arbitrary",)),
    )(mean1, invstd1, xf, e1, xf, e2, w1mat, bias1, w0mat, bias0,
      w2mat, bias2, u_lo)


def kernel(x, labels, embed1, embed2, w1, b1, w2, b2, w0, b0):
    n, c_in, h, w = x.shape
    c_out = w1.shape[0]
    s_lo = h * w

    xf = x.reshape(n, c_in, s_lo)

    e1 = embed1[labels].reshape(n, 2 * c_in, 1)
    e2 = embed2[labels].reshape(n, 2 * c_out, 1)

    w1mat = (jnp.transpose(w1, (0, 2, 3, 1)).reshape(c_out, 9 * c_in)
             .astype(jnp.bfloat16))
    w2mat = (jnp.transpose(w2, (0, 2, 3, 1)).reshape(c_out, 9 * c_out)
             .astype(jnp.bfloat16))
    w0mat = w0.reshape(c_out, c_in).astype(jnp.bfloat16)
    bias1 = b1.reshape(c_out, 1)
    bias2 = b2.reshape(c_out, 1)
    bias0 = b0.reshape(c_out, 1)

    u_lo = _upsample_onehot(w, 2 * w, s_lo, 4 * s_lo, jnp.bfloat16)

    if n % 8 == 0:
        nb1, nb2 = 4, 4
    else:
        nb1, nb2 = 1, 1
    mean1, invstd1 = _bn1_stats(xf, n=n, c_in=c_in, s_lo=s_lo)
    out = _fused(xf, mean1, invstd1, e1, e2, w1mat, bias1, w0mat, bias0,
                 w2mat, bias2, u_lo,
                 n=n, c_in=c_in, c_out=c_out, h=h, w=w, nb1=nb1, nb2=nb2)
    return out.reshape(n, c_out, 2 * h, 2 * w)
